# baseline (speedup 1.0000x reference)
import jax
import jax.numpy as jnp
from jax.experimental import pallas as pl
from jax.experimental.pallas import tpu as pltpu

_GRID = 25


def _copy_body(xb, eb, xob, eob):
    xob[...] = xb[...]
    eob[...] = eb[...]


def kernel(x, edge_index, edge_attr):
    del edge_index
    xf = x.reshape(-1)
    ef = edge_attr.reshape(-1)
    xc = xf.shape[0] // _GRID
    ec = ef.shape[0] // _GRID
    xo, eo = pl.pallas_call(
        _copy_body,
        grid=(_GRID,),
        in_specs=[
            pl.BlockSpec((xc,), lambda i: (i,)),
            pl.BlockSpec((ec,), lambda i: (i,)),
        ],
        out_specs=[
            pl.BlockSpec((xc,), lambda i: (i,)),
            pl.BlockSpec((ec,), lambda i: (i,)),
        ],
        out_shape=[
            jax.ShapeDtypeStruct(xf.shape, xf.dtype),
            jax.ShapeDtypeStruct(ef.shape, ef.dtype),
        ],
    )(xf, ef)
    return (xo.reshape(x.shape), eo.reshape(edge_attr.shape))


# speedup vs baseline: 1.1311x; 1.1311x over previous
import jax
import jax.numpy as jnp
from jax.experimental import pallas as pl

_GRID = 25


def _copy_body(xb, eb, xob, eob):
    xob[...] = xb[...]
    eob[...] = eb[...]


def kernel(x, edge_index, edge_attr):
    del edge_index
    xb = x.shape[0] // _GRID
    eb = edge_attr.shape[0] // _GRID
    xo, eo = pl.pallas_call(
        _copy_body,
        grid=(_GRID,),
        in_specs=[
            pl.BlockSpec((xb, x.shape[1]), lambda i: (i, 0)),
            pl.BlockSpec((eb, edge_attr.shape[1]), lambda i: (i, 0)),
        ],
        out_specs=[
            pl.BlockSpec((xb, x.shape[1]), lambda i: (i, 0)),
            pl.BlockSpec((eb, edge_attr.shape[1]), lambda i: (i, 0)),
        ],
        out_shape=[
            jax.ShapeDtypeStruct(x.shape, x.dtype),
            jax.ShapeDtypeStruct(edge_attr.shape, edge_attr.dtype),
        ],
    )(x, edge_attr)
    return (xo, eo)
